# baseline (speedup 1.0000x reference)
import functools

import jax
import jax.numpy as jnp
from jax import lax
from jax.experimental import pallas as pl
from jax.experimental.pallas import tpu as pltpu
from jax.experimental.pallas import tpu_sc as plsc

N_FIELDS = 26
VOCAB = 100000
DIM = 128
BATCH = 16384

NC = 2
NS = 16
NW = NC * NS
BPW = BATCH // NW
SUB = 8
NB = BPW // SUB
NCHUNK = N_FIELDS * SUB

NBUF = 8
NOUTER = NCHUNK // NBUF


def _embed_kernel(table_hbm, idx_hbm, out_hbm, idx_src, idx_all, rows,
                  g_sems, w_sems):
    wid = lax.axis_index("c") * NS + lax.axis_index("s")
    base = wid * BPW

    pltpu.sync_copy(idx_hbm.at[:, pl.ds(base, BPW)], idx_src)

    def compute_chunk(g):
        f = g // SUB
        s = g % SUB
        for j in range(NB // 16):
            idx_all[g, pl.ds(j * 16, 16)] = (
                idx_src[f, pl.ds(s * NB + j * 16, 16)] + 1)

    def gather(g, b):
        f = g // SUB
        return pltpu.make_async_copy(
            table_hbm.at[f].at[idx_all.at[g]], rows.at[b], g_sems.at[b])

    def write(g, b):
        f = g // SUB
        s = g % SUB
        return pltpu.make_async_copy(
            rows.at[b],
            out_hbm.at[pl.ds(base + s * NB, NB), pl.ds(f * DIM, DIM)],
            w_sems.at[b])

    for b in range(NBUF):
        compute_chunk(b)
    for b in range(NBUF):
        gather(b, b).start()

    def group_body(i, _):
        for b in range(NBUF):
            compute_chunk((i + 1) * NBUF + b)
        for b in range(NBUF):
            g = i * NBUF + b
            gather(g, b).wait()
            write(g, b).start()
        for b in range(NBUF):
            g = i * NBUF + b
            write(g, b).wait()
            gather(g + NBUF, b).start()
        return 0

    lax.fori_loop(0, NOUTER - 1, group_body, 0)

    for b in range(NBUF):
        g = (NOUTER - 1) * NBUF + b
        gather(g, b).wait()
        write(g, b).start()
    for b in range(NBUF):
        g = (NOUTER - 1) * NBUF + b
        write(g, b).wait()


@jax.jit
def _embed(tables, indices):
    k = functools.partial(
        pl.kernel,
        mesh=plsc.VectorSubcoreMesh(core_axis_name="c", subcore_axis_name="s"),
        out_type=jax.ShapeDtypeStruct((BATCH, N_FIELDS * DIM), jnp.float32),
        scratch_types=[
            pltpu.VMEM((N_FIELDS, BPW), jnp.int32),
            pltpu.VMEM((NCHUNK, NB), jnp.int32),
            pltpu.VMEM((NBUF, NB, DIM), jnp.float32),
            pltpu.SemaphoreType.DMA((NBUF,)),
            pltpu.SemaphoreType.DMA((NBUF,)),
        ],
    )(_embed_kernel)
    return k(tables, indices)


def kernel(indices, tables):
    return _embed(tables, indices)


# speedup vs baseline: 1.0256x; 1.0256x over previous
import functools

import jax
import jax.numpy as jnp
from jax import lax
from jax.experimental import pallas as pl
from jax.experimental.pallas import tpu as pltpu
from jax.experimental.pallas import tpu_sc as plsc

N_FIELDS = 26
VOCAB = 100000
DIM = 128
BATCH = 16384

NC = 2
NS = 16
NW = NC * NS
BPW = BATCH // NW
SUB = 8
NB = BPW // SUB
NCHUNK = N_FIELDS * SUB

NBUF = 4
SBUF = 4
NOUTER = NCHUNK // NBUF


def _embed_kernel(table_hbm, idx_hbm, out_hbm, idx_src, idx_all, rows,
                  shared, g_sems, c_sems, w_sems):
    sid = lax.axis_index("s")
    wid = lax.axis_index("c") * NS + sid
    base = wid * BPW

    pltpu.sync_copy(idx_hbm.at[:, pl.ds(base, BPW)], idx_src)

    def compute_chunk(g):
        f = g // SUB
        s = g % SUB
        for j in range(NB // 16):
            idx_all[g, pl.ds(j * 16, 16)] = (
                idx_src[f, pl.ds(s * NB + j * 16, 16)] + 1)

    def gather(g, b):
        f = g // SUB
        return pltpu.make_async_copy(
            table_hbm.at[f].at[idx_all.at[g]], rows.at[b], g_sems.at[b])

    def stage(b):
        return pltpu.make_async_copy(
            rows.at[b], shared.at[sid, b % SBUF], c_sems.at[b])

    def write(g, b):
        f = g // SUB
        s = g % SUB
        return pltpu.make_async_copy(
            shared.at[sid, b % SBUF],
            out_hbm.at[pl.ds(base + s * NB, NB), pl.ds(f * DIM, DIM)],
            w_sems.at[b])

    for b in range(NBUF):
        compute_chunk(b)
    for b in range(NBUF):
        gather(b, b).start()

    for b in range(NBUF):
        compute_chunk(NBUF + b)
    for b in range(NBUF):
        gather(b, b).wait()
        stage(b).start()
    for b in range(NBUF):
        stage(b).wait()
        write(b, b).start()
        gather(b + NBUF, b).start()

    def group_body1(i_, _):
        i = i_ + 1
        for b in range(NBUF):
            compute_chunk((i + 1) * NBUF + b)
        for b in range(NBUF):
            g = i * NBUF + b
            gather(g, b).wait()
            stage(b).start()
        for b in range(NBUF):
            g = i * NBUF + b
            write(g - NBUF, b).wait()
            stage(b).wait()
            write(g, b).start()
            gather(g + NBUF, b).start()
        return 0

    lax.fori_loop(0, NOUTER - 2, group_body1, 0)

    for b in range(NBUF):
        g = (NOUTER - 1) * NBUF + b
        gather(g, b).wait()
        stage(b).start()
    for b in range(NBUF):
        g = (NOUTER - 1) * NBUF + b
        write(g - NBUF, b).wait()
        stage(b).wait()
        write(g, b).start()
    for b in range(NBUF):
        g = (NOUTER - 1) * NBUF + b
        write(g, b).wait()


@jax.jit
def _embed(tables, indices):
    k = functools.partial(
        pl.kernel,
        mesh=plsc.VectorSubcoreMesh(core_axis_name="c", subcore_axis_name="s"),
        out_type=jax.ShapeDtypeStruct((BATCH, N_FIELDS * DIM), jnp.float32),
        scratch_types=[
            pltpu.VMEM((N_FIELDS, BPW), jnp.int32),
            pltpu.VMEM((NCHUNK, NB), jnp.int32),
            pltpu.VMEM((NBUF, NB, DIM), jnp.float32),
            pltpu.VMEM_SHARED((NS, SBUF, NB, DIM), jnp.float32),
            pltpu.SemaphoreType.DMA((NBUF,)),
            pltpu.SemaphoreType.DMA((NBUF,)),
            pltpu.SemaphoreType.DMA((NBUF,)),
        ],
    )(_embed_kernel)
    return k(tables, indices)


def kernel(indices, tables):
    return _embed(tables, indices)
